# Initial kernel scaffold; baseline (speedup 1.0000x reference)
import jax
import jax.numpy as jnp
from jax.experimental import pallas as pl


def kernel(x, old_edge_index, W_gcn, b_gcn, W_graph, b_graph, W_final, b_final, node_scales, graph_scales, graph_means, final_scales, final_means):
    raise NotImplementedError("write your pallas kernel here")


# speedup vs baseline: 18.2747x; 18.2747x over previous
import functools

import jax
import jax.numpy as jnp
from jax import lax
from jax.experimental import pallas as pl
from jax.experimental.pallas import tpu as pltpu
from jax.experimental.pallas import tpu_sc as plsc

N = 10000
F = 128
G = 40
NPG = 250
E = 320000
NW = 32
EPW = 10240
E_PAD = NW * EPW
K = 128
NCH = EPW // K
VROWS = N * G // 16


def _tc_body(x_ref, wgcn_ref, wgraph_ref, wfinal_ref, bgcn_ref, bgraph_ref,
             consts_ref, h_ref, v_ref, p_ref):
    inv = 1.0 / consts_ref[0, 0]
    h = lax.dot_general(x_ref[...], wgcn_ref[...], (((1,), (0,)), ((), ())),
                        preferred_element_type=jnp.float32) * inv
    h_ref[...] = h
    v = lax.dot_general(wfinal_ref[...], wgraph_ref[...],
                        (((1,), (0,)), ((), ())),
                        preferred_element_type=jnp.float32)
    v_ref[...] = v
    rows = lax.broadcasted_iota(jnp.int32, (N, G), 0)
    cols = lax.broadcasted_iota(jnp.int32, (N, G), 1)
    mask = (rows >= cols * NPG) & (rows < (cols + 1) * NPG)
    hb = h + jnp.reshape(bgcn_ref[...], (1, G))
    sel = jnp.sum(jnp.where(mask, hb, 0.0), axis=1, keepdims=True)
    t12 = lax.dot_general(v, sel, (((1,), (0,)), ((), ())),
                          preferred_element_type=jnp.float32)
    wf = wfinal_ref[...]
    const2 = (jnp.sum(wf * jnp.reshape(bgraph_ref[...], (1, G)))
              - consts_ref[0, 1] * jnp.sum(wf))
    p_ref[...] = t12 + const2


_tc_call = pl.pallas_call(
    _tc_body,
    out_shape=(
        jax.ShapeDtypeStruct((N, G), jnp.float32),
        jax.ShapeDtypeStruct((1, N), jnp.float32),
        jax.ShapeDtypeStruct((1, 1), jnp.float32),
    ),
)


def _sc_body(hflat_hbm, src_hbm, dst_hbm, vext_hbm, out_hbm,
             src_v, dst_v, row_v, slab_h, slab_w, acc_v, sem, sem2):
    wid = lax.axis_index("s") * 2 + lax.axis_index("c")
    pltpu.sync_copy(src_hbm.at[pl.ds(wid * EPW, EPW)], src_v)
    pltpu.sync_copy(dst_hbm.at[pl.ds(wid * NCH, NCH)], dst_v)

    def idx_body(i, carry):
        j = lax.shift_right_logical(i, 3)
        o = lax.bitwise_and(i, 7) * 16
        s16 = src_v[pl.ds(i * 16, 16)]
        d16 = dst_v[j, pl.ds(o, 16)]
        c16 = lax.shift_right_logical(d16 * 8389, 21)
        row_v[j, pl.ds(o, 16)] = s16 * G + c16
        return carry

    lax.fori_loop(0, EPW // 16, idx_body, 0)

    def chunk_body(j, acc):
        cp1 = pltpu.async_copy(hflat_hbm.at[row_v.at[j]], slab_h, sem)
        cp2 = pltpu.async_copy(vext_hbm.at[dst_v.at[j]], slab_w, sem2)
        cp1.wait()
        cp2.wait()

        def grp(i, a):
            return a + slab_h[pl.ds(i * 16, 16)] * slab_w[pl.ds(i * 16, 16)]

        return lax.fori_loop(0, K // 16, grp, acc)

    acc = lax.fori_loop(0, NCH, chunk_body, jnp.zeros((16,), jnp.float32))
    acc_v[...] = acc
    pltpu.sync_copy(acc_v, out_hbm.at[wid])


@functools.cache
def _get_sc_call():
    return functools.partial(
        pl.kernel,
        mesh=plsc.VectorSubcoreMesh(core_axis_name="c", subcore_axis_name="s"),
        out_type=jax.ShapeDtypeStruct((NW, 16), jnp.float32),
        scratch_types=[
            pltpu.VMEM((EPW,), jnp.int32),
            pltpu.VMEM((NCH, K), jnp.int32),
            pltpu.VMEM((NCH, K), jnp.int32),
            pltpu.VMEM((K,), jnp.float32),
            pltpu.VMEM((K,), jnp.float32),
            pltpu.VMEM((16,), jnp.float32),
            pltpu.SemaphoreType.DMA,
            pltpu.SemaphoreType.DMA,
        ],
    )(_sc_body)


def kernel(x, old_edge_index, W_gcn, b_gcn, W_graph, b_graph, W_final,
           b_final, node_scales, graph_scales, graph_means, final_scales,
           final_means):
    consts = jnp.stack([jnp.asarray(node_scales, jnp.float32),
                        jnp.asarray(final_means, jnp.float32)]).reshape(1, 2)
    h, v, p = _tc_call(x, W_gcn, W_graph, W_final, b_gcn, b_graph, consts)

    h16 = jnp.reshape(h, (N * G,))
    v_ext = jnp.concatenate([jnp.reshape(v, (N,)),
                             jnp.zeros((16,), jnp.float32)])
    pad = E_PAD - E
    src_pad = jnp.concatenate([old_edge_index[0],
                               jnp.zeros((pad,), old_edge_index.dtype)])
    dst_pad = jnp.concatenate([old_edge_index[1],
                               jnp.full((pad,), N, old_edge_index.dtype)])
    dst_pad = jnp.reshape(dst_pad, (NW * NCH, K))

    partials = _get_sc_call()(h16, src_pad, dst_pad, v_ext)
    s = p[0, 0] + jnp.sum(partials)
    y = jax.nn.sigmoid(s / final_scales + b_final[0])
    return jnp.reshape(y, (1,))


# speedup vs baseline: 22.0718x; 1.2078x over previous
import functools

import jax
import jax.numpy as jnp
from jax import lax
from jax.experimental import pallas as pl
from jax.experimental.pallas import tpu as pltpu
from jax.experimental.pallas import tpu_sc as plsc

N = 10000
F = 128
G = 40
NPG = 250
E = 320000
NW = 32
EPW = 10240
E_PAD = NW * EPW
K = 128
NCH = EPW // K
VROWS = N * G // 16


def _tc_body(x_ref, wgcn_ref, wgraph_ref, wfinal_ref, bgcn_ref, bgraph_ref,
             consts_ref, h_ref, v_ref, p_ref):
    inv = 1.0 / consts_ref[0, 0]
    h = lax.dot_general(x_ref[...], wgcn_ref[...], (((1,), (0,)), ((), ())),
                        preferred_element_type=jnp.float32) * inv
    h_ref[...] = h
    v = lax.dot_general(wfinal_ref[...], wgraph_ref[...],
                        (((1,), (0,)), ((), ())),
                        preferred_element_type=jnp.float32)
    v_ref[...] = v
    rows = lax.broadcasted_iota(jnp.int32, (N, G), 0)
    cols = lax.broadcasted_iota(jnp.int32, (N, G), 1)
    mask = (rows >= cols * NPG) & (rows < (cols + 1) * NPG)
    hb = h + jnp.reshape(bgcn_ref[...], (1, G))
    sel = jnp.sum(jnp.where(mask, hb, 0.0), axis=1, keepdims=True)
    t12 = lax.dot_general(v, sel, (((1,), (0,)), ((), ())),
                          preferred_element_type=jnp.float32)
    wf = wfinal_ref[...]
    const2 = (jnp.sum(wf * jnp.reshape(bgraph_ref[...], (1, G)))
              - consts_ref[0, 1] * jnp.sum(wf))
    p_ref[...] = t12 + const2


_tc_call = pl.pallas_call(
    _tc_body,
    out_shape=(
        jax.ShapeDtypeStruct((N, G), jnp.float32),
        jax.ShapeDtypeStruct((1, N), jnp.float32),
        jax.ShapeDtypeStruct((1, 1), jnp.float32),
    ),
)


RING = 2


def _sc_body(hflat_hbm, src_hbm, dst_hbm, vext_hbm, out_hbm,
             src_v, dst_v, row_v, slab_h, slab_w, acc_v, *sems):
    wid = lax.axis_index("s") * 2 + lax.axis_index("c")
    pltpu.sync_copy(src_hbm.at[pl.ds(wid * EPW, EPW)], src_v)
    pltpu.sync_copy(dst_hbm.at[pl.ds(wid * NCH, NCH)], dst_v)

    def idx_body(i, carry):
        j = lax.shift_right_logical(i, 3)
        o = lax.bitwise_and(i, 7) * 16
        s16 = src_v[pl.ds(i * 16, 16)]
        d16 = dst_v[j, pl.ds(o, 16)]
        c16 = lax.shift_right_logical(d16 * 8389, 21)
        row_v[j, pl.ds(o, 16)] = s16 * G + c16
        return carry

    lax.fori_loop(0, K // 16 * NCH, idx_body, 0)

    def issue(j, b):
        pltpu.async_copy(hflat_hbm.at[row_v.at[j]], slab_h.at[b], sems[b])
        pltpu.async_copy(vext_hbm.at[dst_v.at[j]], slab_w.at[b],
                         sems[RING + b])

    def drain(j, b):
        pltpu.make_async_copy(hflat_hbm.at[row_v.at[j]], slab_h.at[b],
                              sems[b]).wait()
        pltpu.make_async_copy(vext_hbm.at[dst_v.at[j]], slab_w.at[b],
                              sems[RING + b]).wait()

    for b in range(RING):
        issue(b, b)

    @pl.loop(0, NCH, step=RING, init_carry=jnp.zeros((16,), jnp.float32))
    def outer(j2, acc):
        for b in range(RING):
            j = j2 + b
            drain(j, b)

            def grp(i, a):
                return (a + slab_h[b, pl.ds(i * 16, 16)]
                        * slab_w[b, pl.ds(i * 16, 16)])

            acc = lax.fori_loop(0, K // 16, grp, acc)

            @pl.when(j + RING < NCH)
            def _():
                issue(j + RING, b)
        return acc

    acc_v[...] = outer
    pltpu.sync_copy(acc_v, out_hbm.at[wid])


@functools.cache
def _get_sc_call():
    return functools.partial(
        pl.kernel,
        mesh=plsc.VectorSubcoreMesh(core_axis_name="c", subcore_axis_name="s"),
        out_type=jax.ShapeDtypeStruct((NW, 16), jnp.float32),
        scratch_types=[
            pltpu.VMEM((EPW,), jnp.int32),
            pltpu.VMEM((NCH, K), jnp.int32),
            pltpu.VMEM((NCH, K), jnp.int32),
            pltpu.VMEM((RING, K), jnp.float32),
            pltpu.VMEM((RING, K), jnp.float32),
            pltpu.VMEM((16,), jnp.float32),
        ] + [pltpu.SemaphoreType.DMA] * (2 * RING),
    )(_sc_body)


def kernel(x, old_edge_index, W_gcn, b_gcn, W_graph, b_graph, W_final,
           b_final, node_scales, graph_scales, graph_means, final_scales,
           final_means):
    consts = jnp.stack([jnp.asarray(node_scales, jnp.float32),
                        jnp.asarray(final_means, jnp.float32)]).reshape(1, 2)
    h, v, p = _tc_call(x, W_gcn, W_graph, W_final, b_gcn, b_graph, consts)

    h16 = jnp.reshape(h, (N * G,))
    v_ext = jnp.concatenate([jnp.reshape(v, (N,)),
                             jnp.zeros((16,), jnp.float32)])
    pad = E_PAD - E
    src_pad = jnp.concatenate([old_edge_index[0],
                               jnp.zeros((pad,), old_edge_index.dtype)])
    dst_pad = jnp.concatenate([old_edge_index[1],
                               jnp.full((pad,), N, old_edge_index.dtype)])
    dst_pad = jnp.reshape(dst_pad, (NW * NCH, K))

    partials = _get_sc_call()(h16, src_pad, dst_pad, v_ext)
    s = p[0, 0] + jnp.sum(partials)
    y = jax.nn.sigmoid(s / final_scales + b_final[0])
    return jnp.reshape(y, (1,))
